# baseline (speedup 1.0000x reference)
import functools

import jax
import jax.numpy as jnp
from jax import lax
from jax.experimental import pallas as pl
from jax.experimental.pallas import tpu as pltpu
from jax.experimental.pallas import tpu_sc as plsc

N = 10000
NPAD = 10016
E = 160000
EP = 163840
D = 256
HF = 128
S = 5
TAU = 0.5
LAM = 0.1
EPS = 1e-5
NACC = 10112
NA = 10240
P = NA * 2 * S
BLK = 512
GRID = 20

_MESH = plsc.VectorSubcoreMesh(core_axis_name="c", subcore_axis_name="s")


def _deg_body(srcd, dstd, zflat, out_s, out_d, idxv, histv):
    c = lax.axis_index("c")
    s = lax.axis_index("s")
    w = s * 2 + c
    ones = jnp.full((16,), 1.0, jnp.float32)

    def one_pass(edges_hbm, out_hbm):
        pltpu.sync_copy(zflat, histv)
        pltpu.sync_copy(edges_hbm.at[pl.ds(w * 5120, 5120)], idxv)

        def chunk(j, _):
            base = pl.multiple_of(j * 128, 128)
            for g in range(8):
                idx16 = idxv[pl.ds(base + g * 16, 16)]
                plsc.addupdate_scatter(histv, [idx16], ones)
            return ()

        lax.fori_loop(0, 40, chunk, ())
        pltpu.sync_copy(histv, out_hbm.at[pl.ds(w * 16384, 16384)])

    one_pass(srcd, out_s)
    one_pass(dstd, out_d)


_deg_call = pl.kernel(
    _deg_body,
    out_type=[
        jax.ShapeDtypeStruct((32 * 16384,), jnp.float32),
        jax.ShapeDtypeStruct((32 * 16384,), jnp.float32),
    ],
    mesh=_MESH,
    scratch_types=[
        pltpu.VMEM((5120,), jnp.int32),
        pltpu.VMEM((16384,), jnp.float32),
    ],
    compiler_params=pltpu.CompilerParams(needs_layout_passes=False),
)


def _norm_body(sref, dref, oref):
    sh = jnp.sum(sref[...].reshape(32, 128, 128), axis=0)
    dh = jnp.sum(dref[...].reshape(32, 128, 128), axis=0)
    oref[0] = lax.rsqrt(jnp.clip(sh, 1.0, None))
    oref[1] = lax.rsqrt(jnp.clip(dh, 1.0, None))


def _norms(hs, hd):
    return pl.pallas_call(
        _norm_body,
        out_shape=jax.ShapeDtypeStruct((2, 128, 128), jnp.float32),
    )(hs.reshape(4096, 128), hd.reshape(4096, 128))


def _rowscale(v, nref, which, i):
    r = BLK // 128
    t = nref[which, pl.ds(i * r, r), :]
    return (v.reshape(r, 128, D) * t[..., None]).reshape(BLK, D)


def _mm1_body(xref, nref, wref, oref):
    i = pl.program_id(0)
    res = jnp.dot(_rowscale(xref[...], nref, 0, i), wref[...],
                  preferred_element_type=jnp.float32)
    oref[0] = res[:, :HF]
    oref[1] = res[:, HF:]


def _mm1(x, norms, W0):
    return pl.pallas_call(
        _mm1_body,
        grid=(GRID,),
        in_specs=[
            pl.BlockSpec((BLK, D), lambda i: (i, 0)),
            pl.BlockSpec((2, 128, 128), lambda i: (0, 0, 0)),
            pl.BlockSpec((D, D), lambda i: (0, 0)),
        ],
        out_specs=pl.BlockSpec((2, BLK, HF), lambda i: (0, i, 0)),
        out_shape=jax.ShapeDtypeStruct((2, NPAD, HF), jnp.float32),
    )(x, norms, W0)


def _mm2_body(aref, nref, bref, wref, oref):
    i = pl.program_id(0)
    h = jnp.concatenate([aref[0], aref[1]], axis=-1)
    h = _rowscale(h, nref, 1, i) + bref[...]
    h = _rowscale(h, nref, 0, i)
    res = jnp.dot(h, wref[...], preferred_element_type=jnp.float32)
    oref[0] = res[:, :HF]
    oref[1] = res[:, HF:]


def _mm2(aggf, norms, b0, W1):
    return pl.pallas_call(
        _mm2_body,
        grid=(GRID,),
        in_specs=[
            pl.BlockSpec((2, BLK, HF), lambda i: (0, i, 0)),
            pl.BlockSpec((2, 128, 128), lambda i: (0, 0, 0)),
            pl.BlockSpec((1, D), lambda i: (0, 0)),
            pl.BlockSpec((D, D), lambda i: (0, 0)),
        ],
        out_specs=pl.BlockSpec((2, BLK, HF), lambda i: (0, i, 0)),
        out_shape=jax.ShapeDtypeStruct((2, NPAD, HF), jnp.float32),
    )(aggf.reshape(2, NACC, HF), norms, b0.reshape(1, D), W1)


def _bn_body(xref, wref, oref):
    i = pl.program_id(0)
    u = jnp.dot(xref[...], wref[...], preferred_element_type=jnp.float32)
    rowmask = (i * BLK + lax.broadcasted_iota(jnp.int32, (BLK, 1), 0)) < N
    u = jnp.where(rowmask, u, 0.0)

    @pl.when(i == 0)
    def _():
        oref[...] = jnp.zeros_like(oref)

    oref[0:1, :] = oref[0:1, :] + jnp.sum(u, axis=0)[None, :]
    oref[1:2, :] = oref[1:2, :] + jnp.sum(u * u, axis=0)[None, :]


def _bn_stats(x, Wt1):
    return pl.pallas_call(
        _bn_body,
        grid=(GRID,),
        in_specs=[
            pl.BlockSpec((BLK, D), lambda i: (i, 0)),
            pl.BlockSpec((D, D), lambda i: (0, 0)),
        ],
        out_specs=pl.BlockSpec((8, D), lambda i: (0, 0)),
        out_shape=jax.ShapeDtypeStruct((8, D), jnp.float32),
    )(x, Wt1)


def _l2n(v):
    ss = jnp.sum(v * v, axis=-1, keepdims=True)
    return v * lax.rsqrt(jnp.maximum(ss, 1e-24))


def _gnn_body(aref, nref, b1ref, gout):
    i = pl.program_id(0)
    g = jnp.concatenate([aref[0], aref[1]], axis=-1)
    g = _rowscale(g, nref, 1, i) + b1ref[...]
    gout[...] = _l2n(g)


def _gnn(aggf, norms, b1):
    return pl.pallas_call(
        _gnn_body,
        grid=(GRID,),
        in_specs=[
            pl.BlockSpec((2, BLK, HF), lambda i: (0, i, 0)),
            pl.BlockSpec((2, 128, 128), lambda i: (0, 0, 0)),
            pl.BlockSpec((1, D), lambda i: (0, 0)),
        ],
        out_specs=pl.BlockSpec((BLK, D), lambda i: (i, 0)),
        out_shape=jax.ShapeDtypeStruct((N, D), jnp.float32),
    )(aggf.reshape(2, NACC, HF), norms, b1.reshape(1, D))


def _mlp_body(stref, xref, wt1, bt1r, gr, br, wt2, bt2r,
              pw0, pb0, pw1, pb1, pw2, pb2, mout, pout):
    u = jnp.dot(xref[...], wt1[...], preferred_element_type=jnp.float32)
    s1 = stref[0:1, :]
    s2 = stref[1:2, :]
    m = s1 * (1.0 / N)
    var = s2 * (1.0 / N) - m * m
    t = (u - m) * lax.rsqrt(var + EPS) * gr[...] + br[...]
    t = jnp.maximum(t, 0.0)
    t = jnp.dot(t, wt2[...], preferred_element_type=jnp.float32) + bt2r[...]
    mlp = _l2n(t)
    mout[...] = mlp

    p = jnp.maximum(jnp.dot(mlp, pw0[...], preferred_element_type=jnp.float32) + pb0[...], 0.0)
    p = jnp.maximum(jnp.dot(p, pw1[...], preferred_element_type=jnp.float32) + pb1[...], 0.0)
    p = jnp.dot(p, pw2[...], preferred_element_type=jnp.float32) + pb2[...]
    pout[...] = _l2n(p)


def _mlp(stats, x, Wt1, bt1, gamma, beta, Wt2, bt2,
         PW0, Pb0, PW1, Pb1, PW2, Pb2):
    row = lambda i: (i, 0)
    full = lambda i: (0, 0)
    vec = pl.BlockSpec((1, D), full)
    mat = pl.BlockSpec((D, D), full)
    return pl.pallas_call(
        _mlp_body,
        grid=(GRID,),
        in_specs=[
            pl.BlockSpec((8, D), full),
            pl.BlockSpec((BLK, D), row),
            mat, vec, vec, vec, mat, vec,
            mat, vec, mat, vec, mat, vec,
        ],
        out_specs=[
            pl.BlockSpec((BLK, D), row),
            pl.BlockSpec((BLK, D), row),
        ],
        out_shape=[
            jax.ShapeDtypeStruct((N, D), jnp.float32),
            jax.ShapeDtypeStruct((N, D), jnp.float32),
        ],
    )(stats, x,
      Wt1, bt1.reshape(1, D), gamma.reshape(1, D), beta.reshape(1, D),
      Wt2, bt2.reshape(1, D), PW0, Pb0.reshape(1, D), PW1, Pb1.reshape(1, D),
      PW2, Pb2.reshape(1, D))


def _scat_body(y_hbm, sab_hbm, dst_hbm, z_hbm, out_hbm, sidx, didx,
               gbuf0, gbuf1, acc, gsem0, gsem1):
    c = lax.axis_index("c")
    s = lax.axis_index("s")
    pltpu.sync_copy(z_hbm.at[pl.ds(s * 632, 632)], acc.at[pl.ds(s * 632, 632)])
    plsc.subcore_barrier()

    def gat(j, buf, sem):
        return pltpu.make_async_copy(y_hbm.at[sidx.at[j]], buf, sem)

    def phase(ph, _):
        pltpu.sync_copy(sab_hbm.at[pl.ds(c * 1280 + s * 80 + ph * 40, 40)], sidx)
        pltpu.sync_copy(dst_hbm.at[pl.ds(s * 80 + ph * 40, 40)], didx)
        gat(0, gbuf0, gsem0).start()

        def chunk2(jj, _):
            j = jj * 2
            gat(j + 1, gbuf1, gsem1).start()
            gat(j, gbuf0, gsem0).wait()
            pltpu.sync_copy(gbuf0, acc.at[didx.at[j]], add=True)

            @pl.when(jj < 19)
            def _():
                gat(j + 2, gbuf0, gsem0).start()

            gat(j + 1, gbuf1, gsem1).wait()
            pltpu.sync_copy(gbuf1, acc.at[didx.at[j + 1]], add=True)
            return ()

        lax.fori_loop(0, 20, chunk2, ())
        return ()

    lax.fori_loop(0, 2, phase, ())
    plsc.subcore_barrier()
    pltpu.sync_copy(acc.at[pl.ds(s * 632, 632)],
                    out_hbm.at[pl.ds(c * NACC + s * 632, 632)])


_scat_call = pl.kernel(
    _scat_body,
    out_type=jax.ShapeDtypeStruct((2 * NACC, HF), jnp.float32),
    mesh=_MESH,
    scratch_types=[
        pltpu.VMEM((40, 128), jnp.int32),
        pltpu.VMEM((40, 128), jnp.int32),
        pltpu.VMEM((128, HF), jnp.float32),
        pltpu.VMEM((128, HF), jnp.float32),
        pltpu.VMEM_SHARED((NACC, HF), jnp.float32),
        pltpu.SemaphoreType.DMA,
        pltpu.SemaphoreType.DMA,
    ],
)


_LGC0 = 20
_LGC1 = 50 - _LGC0


def _lg_body(g_hbm, gi_hbm, out_hbm, giv, gbuf0, gbuf1, gbuf2,
             gs0, gs1, gs2, ws0, ws1, ws2):
    c = lax.axis_index("c")
    s = lax.axis_index("s")

    bufs = (gbuf0, gbuf1, gbuf2)
    gsems = (gs0, gs1, gs2)
    wsems = (ws0, ws1, ws2)

    def run(base, nchunk):
        pltpu.sync_copy(gi_hbm.at[pl.ds(base, nchunk * 128)],
                        giv.at[pl.ds(0, nchunk * 128)])

        def gat(k, b):
            return pltpu.make_async_copy(g_hbm.at[giv.at[pl.ds(k * 128, 128)]],
                                         bufs[b], gsems[b])

        def wr(k, b):
            return pltpu.make_async_copy(
                bufs[b], out_hbm.at[pl.ds(base + k * 128, 128)], wsems[b])

        for b in range(3):
            gat(b, b).start()

        nfull = nchunk // 3

        def chunk3(kk, _):
            k = kk * 3
            for b in range(3):
                gat(k + b, b).wait()
                wr(k + b, b).start()

                @pl.when(kk < nfull - 1)
                def _():
                    wr(k + b, b).wait()
                    gat(k + b + 3, b).start()
            return ()

        lax.fori_loop(0, nfull, chunk3, ())
        for r in range(nchunk - nfull * 3):
            k = nfull * 3 + r
            wr(k - 3, r).wait()
            gat(k, r).start()
            gat(k, r).wait()
            wr(k, r).start()
            wr(k, r).wait()
        for b in range(nchunk - nfull * 3, 3):
            wr(nfull * 3 - 3 + b, b).wait()

    @pl.when(c == 0)
    def _():
        run(s * (_LGC0 * 128), _LGC0)

    @pl.when(c == 1)
    def _():
        run(16 * (_LGC0 * 128) + s * (_LGC1 * 128), _LGC1)


_lg_call = pl.kernel(
    _lg_body,
    out_type=jax.ShapeDtypeStruct((P, D), jnp.float32),
    mesh=_MESH,
    scratch_types=[
        pltpu.VMEM((max(_LGC0, _LGC1) * 128,), jnp.int32),
        pltpu.VMEM((128, D), jnp.float32),
        pltpu.VMEM((128, D), jnp.float32),
        pltpu.VMEM((128, D), jnp.float32),
        pltpu.SemaphoreType.DMA,
        pltpu.SemaphoreType.DMA,
        pltpu.SemaphoreType.DMA,
        pltpu.SemaphoreType.DMA,
        pltpu.SemaphoreType.DMA,
        pltpu.SemaphoreType.DMA,
    ],
)


_LPB = 2560
_LNB = P // _LPB


def _loss_body(gref, pref, oref):
    i = pl.program_id(0)
    nb = _LPB // (2 * S)
    g3 = gref[...].reshape(nb, 2 * S, D)
    d = jnp.sum(g3 * pref[...][:, None, :], axis=-1) * (1.0 / TAU)
    e = jnp.exp(d)
    ps = jnp.sum(e[:, :S], axis=-1)
    ns_ = jnp.sum(e[:, S:], axis=-1)
    term = -jnp.log(ps / (ps + LAM * ns_))
    mask = (i * nb + lax.broadcasted_iota(jnp.int32, (nb,), 0)) < N
    blocksum = jnp.sum(jnp.where(mask, term, 0.0)) * (1.0 / N)

    @pl.when(i == 0)
    def _():
        oref[...] = jnp.zeros_like(oref)

    oref[...] = oref[...] + blocksum.reshape(1, 1)


def _loss(grows, proj):
    return pl.pallas_call(
        _loss_body,
        grid=(_LNB,),
        in_specs=[
            pl.BlockSpec((_LPB, D), lambda i: (i, 0)),
            pl.BlockSpec((_LPB // (2 * S), D), lambda i: (i, 0)),
        ],
        out_specs=pl.BlockSpec((1, 1), lambda i: (0, 0)),
        out_shape=jax.ShapeDtypeStruct((1, 1), jnp.float32),
    )(grows, proj)


def kernel(x, edge_index, pos_idx, neg_idx, W0, b0, W1, b1, Wt1, bt1, gamma,
           beta, Wt2, bt2, PW0, Pb0, PW1, Pb1, PW2, Pb2):
    src = edge_index[0]
    dst = edge_index[1]
    padn = EP - E
    i32 = jnp.int32

    src_p = jnp.concatenate([src, jnp.full((padn,), N, i32)])
    dst_p = jnp.concatenate([dst, jnp.full((padn,), N, i32)])
    src2d = src_p.reshape(1280, 128)
    srcAB = jnp.concatenate([src2d, src2d + NPAD], axis=0)
    dst2d = dst_p.reshape(1280, 128)
    srcdeg = jnp.concatenate([src, jnp.full((padn,), 16383, i32)])
    dstdeg = jnp.concatenate([dst, jnp.full((padn,), 16383, i32)])

    gi = jnp.concatenate([pos_idx, neg_idx], axis=1)
    gi = jnp.concatenate([gi, jnp.zeros((NA - N, 2 * S), i32)], axis=0)
    giflat = gi.reshape(-1)

    zflat = jnp.zeros((16384,), jnp.float32)
    zrows = jnp.zeros((NACC, HF), jnp.float32)

    hs, hd = _deg_call(srcdeg, dstdeg, zflat)
    norms = _norms(hs, hd)

    y1 = _mm1(x, norms, W0)
    agg1 = _scat_call(y1.reshape(2 * NPAD, HF), srcAB, dst2d, zrows)
    y2 = _mm2(agg1, norms, b0, W1)
    agg2 = _scat_call(y2.reshape(2 * NPAD, HF), srcAB, dst2d, zrows)

    stats = _bn_stats(x, Wt1)
    MLP_emb, proj = _mlp(stats, x, Wt1, bt1, gamma, beta, Wt2, bt2,
                         PW0, Pb0, PW1, Pb1, PW2, Pb2)
    GNN_emb = _gnn(agg2, norms, b1)

    grows = _lg_call(GNN_emb, giflat)
    lossm = _loss(grows, proj)
    return (lossm[0, 0], MLP_emb)


# speedup vs baseline: 1.0029x; 1.0029x over previous
import functools

import jax
import jax.numpy as jnp
from jax import lax
from jax.experimental import pallas as pl
from jax.experimental.pallas import tpu as pltpu
from jax.experimental.pallas import tpu_sc as plsc

N = 10000
NPAD = 10016
E = 160000
EP = 163840
D = 256
HF = 128
S = 5
TAU = 0.5
LAM = 0.1
EPS = 1e-5
NACC = 10112
NA = 10240
P = NA * 2 * S
BLK = 512
GRID = 20

_MESH = plsc.VectorSubcoreMesh(core_axis_name="c", subcore_axis_name="s")


def _deg_body(srcd, dstd, zflat, out_s, out_d, idxv, histv):
    c = lax.axis_index("c")
    s = lax.axis_index("s")
    w = s * 2 + c
    ones = jnp.full((16,), 1.0, jnp.float32)

    def one_pass(edges_hbm, out_hbm):
        pltpu.sync_copy(zflat, histv)
        pltpu.sync_copy(edges_hbm.at[pl.ds(w * 5120, 5120)], idxv)

        def chunk(j, _):
            base = pl.multiple_of(j * 128, 128)
            for g in range(8):
                idx16 = idxv[pl.ds(base + g * 16, 16)]
                plsc.addupdate_scatter(histv, [idx16], ones)
            return ()

        lax.fori_loop(0, 40, chunk, ())
        pltpu.sync_copy(histv, out_hbm.at[pl.ds(w * 16384, 16384)])

    one_pass(srcd, out_s)
    one_pass(dstd, out_d)


_deg_call = pl.kernel(
    _deg_body,
    out_type=[
        jax.ShapeDtypeStruct((32 * 16384,), jnp.float32),
        jax.ShapeDtypeStruct((32 * 16384,), jnp.float32),
    ],
    mesh=_MESH,
    scratch_types=[
        pltpu.VMEM((5120,), jnp.int32),
        pltpu.VMEM((16384,), jnp.float32),
    ],
    compiler_params=pltpu.CompilerParams(needs_layout_passes=False),
)


def _norm_body(sref, dref, oref):
    sh = jnp.sum(sref[...].reshape(32, 128, 128), axis=0)
    dh = jnp.sum(dref[...].reshape(32, 128, 128), axis=0)
    oref[0] = lax.rsqrt(jnp.clip(sh, 1.0, None))
    oref[1] = lax.rsqrt(jnp.clip(dh, 1.0, None))


def _norms(hs, hd):
    return pl.pallas_call(
        _norm_body,
        out_shape=jax.ShapeDtypeStruct((2, 128, 128), jnp.float32),
    )(hs.reshape(4096, 128), hd.reshape(4096, 128))


def _rowscale(v, nref, which, i):
    r = BLK // 128
    t = nref[which, pl.ds(i * r, r), :]
    return (v.reshape(r, 128, D) * t[..., None]).reshape(BLK, D)


def _mm1_body(xref, nref, wref, oref):
    i = pl.program_id(0)
    res = jnp.dot(_rowscale(xref[...], nref, 0, i), wref[...],
                  preferred_element_type=jnp.float32)
    oref[0] = res[:, :HF]
    oref[1] = res[:, HF:]


def _mm1(x, norms, W0):
    return pl.pallas_call(
        _mm1_body,
        grid=(GRID,),
        in_specs=[
            pl.BlockSpec((BLK, D), lambda i: (i, 0)),
            pl.BlockSpec((2, 128, 128), lambda i: (0, 0, 0)),
            pl.BlockSpec((D, D), lambda i: (0, 0)),
        ],
        out_specs=pl.BlockSpec((2, BLK, HF), lambda i: (0, i, 0)),
        out_shape=jax.ShapeDtypeStruct((2, NPAD, HF), jnp.float32),
    )(x, norms, W0)


def _mm2_body(aref, nref, bref, wref, oref):
    i = pl.program_id(0)
    h = jnp.concatenate([aref[0], aref[1]], axis=-1)
    h = _rowscale(h, nref, 1, i) + bref[...]
    h = _rowscale(h, nref, 0, i)
    res = jnp.dot(h, wref[...], preferred_element_type=jnp.float32)
    oref[0] = res[:, :HF]
    oref[1] = res[:, HF:]


def _mm2(aggf, norms, b0, W1):
    return pl.pallas_call(
        _mm2_body,
        grid=(GRID,),
        in_specs=[
            pl.BlockSpec((2, BLK, HF), lambda i: (0, i, 0)),
            pl.BlockSpec((2, 128, 128), lambda i: (0, 0, 0)),
            pl.BlockSpec((1, D), lambda i: (0, 0)),
            pl.BlockSpec((D, D), lambda i: (0, 0)),
        ],
        out_specs=pl.BlockSpec((2, BLK, HF), lambda i: (0, i, 0)),
        out_shape=jax.ShapeDtypeStruct((2, NPAD, HF), jnp.float32),
    )(aggf.reshape(2, NACC, HF), norms, b0.reshape(1, D), W1)


def _bn_body(xref, wref, oref):
    i = pl.program_id(0)
    u = jnp.dot(xref[...], wref[...], preferred_element_type=jnp.float32)
    rowmask = (i * BLK + lax.broadcasted_iota(jnp.int32, (BLK, 1), 0)) < N
    u = jnp.where(rowmask, u, 0.0)

    @pl.when(i == 0)
    def _():
        oref[...] = jnp.zeros_like(oref)

    oref[0:1, :] = oref[0:1, :] + jnp.sum(u, axis=0)[None, :]
    oref[1:2, :] = oref[1:2, :] + jnp.sum(u * u, axis=0)[None, :]


def _bn_stats(x, Wt1):
    return pl.pallas_call(
        _bn_body,
        grid=(GRID,),
        in_specs=[
            pl.BlockSpec((BLK, D), lambda i: (i, 0)),
            pl.BlockSpec((D, D), lambda i: (0, 0)),
        ],
        out_specs=pl.BlockSpec((8, D), lambda i: (0, 0)),
        out_shape=jax.ShapeDtypeStruct((8, D), jnp.float32),
    )(x, Wt1)


def _l2n(v):
    ss = jnp.sum(v * v, axis=-1, keepdims=True)
    return v * lax.rsqrt(jnp.maximum(ss, 1e-24))


def _gnn_body(aref, nref, b1ref, gout):
    i = pl.program_id(0)
    g = jnp.concatenate([aref[0], aref[1]], axis=-1)
    g = _rowscale(g, nref, 1, i) + b1ref[...]
    gout[...] = _l2n(g)


def _gnn(aggf, norms, b1):
    return pl.pallas_call(
        _gnn_body,
        grid=(GRID,),
        in_specs=[
            pl.BlockSpec((2, BLK, HF), lambda i: (0, i, 0)),
            pl.BlockSpec((2, 128, 128), lambda i: (0, 0, 0)),
            pl.BlockSpec((1, D), lambda i: (0, 0)),
        ],
        out_specs=pl.BlockSpec((BLK, D), lambda i: (i, 0)),
        out_shape=jax.ShapeDtypeStruct((N, D), jnp.float32),
    )(aggf.reshape(2, NACC, HF), norms, b1.reshape(1, D))


def _mlp_body(stref, xref, wt1, bt1r, gr, br, wt2, bt2r,
              pw0, pb0, pw1, pb1, pw2, pb2, mout, pout):
    u = jnp.dot(xref[...], wt1[...], preferred_element_type=jnp.float32)
    s1 = stref[0:1, :]
    s2 = stref[1:2, :]
    m = s1 * (1.0 / N)
    var = s2 * (1.0 / N) - m * m
    t = (u - m) * lax.rsqrt(var + EPS) * gr[...] + br[...]
    t = jnp.maximum(t, 0.0)
    t = jnp.dot(t, wt2[...], preferred_element_type=jnp.float32) + bt2r[...]
    mlp = _l2n(t)
    mout[...] = mlp

    p = jnp.maximum(jnp.dot(mlp, pw0[...], preferred_element_type=jnp.float32) + pb0[...], 0.0)
    p = jnp.maximum(jnp.dot(p, pw1[...], preferred_element_type=jnp.float32) + pb1[...], 0.0)
    p = jnp.dot(p, pw2[...], preferred_element_type=jnp.float32) + pb2[...]
    pout[...] = _l2n(p)


def _mlp(stats, x, Wt1, bt1, gamma, beta, Wt2, bt2,
         PW0, Pb0, PW1, Pb1, PW2, Pb2):
    row = lambda i: (i, 0)
    full = lambda i: (0, 0)
    vec = pl.BlockSpec((1, D), full)
    mat = pl.BlockSpec((D, D), full)
    return pl.pallas_call(
        _mlp_body,
        grid=(GRID,),
        in_specs=[
            pl.BlockSpec((8, D), full),
            pl.BlockSpec((BLK, D), row),
            mat, vec, vec, vec, mat, vec,
            mat, vec, mat, vec, mat, vec,
        ],
        out_specs=[
            pl.BlockSpec((BLK, D), row),
            pl.BlockSpec((BLK, D), row),
        ],
        out_shape=[
            jax.ShapeDtypeStruct((N, D), jnp.float32),
            jax.ShapeDtypeStruct((N, D), jnp.float32),
        ],
    )(stats, x,
      Wt1, bt1.reshape(1, D), gamma.reshape(1, D), beta.reshape(1, D),
      Wt2, bt2.reshape(1, D), PW0, Pb0.reshape(1, D), PW1, Pb1.reshape(1, D),
      PW2, Pb2.reshape(1, D))


def _scat_body(y_hbm, sab_hbm, dst_hbm, z_hbm, out_hbm, sidx, didx,
               gbuf0, gbuf1, acc, gsem0, gsem1):
    c = lax.axis_index("c")
    s = lax.axis_index("s")
    pltpu.sync_copy(z_hbm.at[pl.ds(s * 632, 632)], acc.at[pl.ds(s * 632, 632)])
    plsc.subcore_barrier()

    def gat(j, buf, sem):
        return pltpu.make_async_copy(y_hbm.at[sidx.at[j]], buf, sem)

    def phase(ph, _):
        pltpu.sync_copy(sab_hbm.at[pl.ds(c * 1280 + s * 80 + ph * 40, 40)], sidx)
        pltpu.sync_copy(dst_hbm.at[pl.ds(s * 80 + ph * 40, 40)], didx)
        gat(0, gbuf0, gsem0).start()

        def chunk2(jj, _):
            j = jj * 2
            gat(j + 1, gbuf1, gsem1).start()
            gat(j, gbuf0, gsem0).wait()
            pltpu.sync_copy(gbuf0, acc.at[didx.at[j]], add=True)

            @pl.when(jj < 19)
            def _():
                gat(j + 2, gbuf0, gsem0).start()

            gat(j + 1, gbuf1, gsem1).wait()
            pltpu.sync_copy(gbuf1, acc.at[didx.at[j + 1]], add=True)
            return ()

        lax.fori_loop(0, 20, chunk2, ())
        return ()

    lax.fori_loop(0, 2, phase, ())
    plsc.subcore_barrier()
    pltpu.sync_copy(acc.at[pl.ds(s * 632, 632)],
                    out_hbm.at[pl.ds(c * NACC + s * 632, 632)])


_scat_call = pl.kernel(
    _scat_body,
    out_type=jax.ShapeDtypeStruct((2 * NACC, HF), jnp.float32),
    mesh=_MESH,
    scratch_types=[
        pltpu.VMEM((40, 128), jnp.int32),
        pltpu.VMEM((40, 128), jnp.int32),
        pltpu.VMEM((128, HF), jnp.float32),
        pltpu.VMEM((128, HF), jnp.float32),
        pltpu.VMEM_SHARED((NACC, HF), jnp.float32),
        pltpu.SemaphoreType.DMA,
        pltpu.SemaphoreType.DMA,
    ],
)


_LGC0 = 30
_LGC1 = 50 - _LGC0


def _lg_body(g_hbm, gi_hbm, out_hbm, giv, gbuf0, gbuf1, gbuf2,
             gs0, gs1, gs2, ws0, ws1, ws2):
    c = lax.axis_index("c")
    s = lax.axis_index("s")

    bufs = (gbuf0, gbuf1, gbuf2)
    gsems = (gs0, gs1, gs2)
    wsems = (ws0, ws1, ws2)

    def run(base, nchunk):
        pltpu.sync_copy(gi_hbm.at[pl.ds(base, nchunk * 128)],
                        giv.at[pl.ds(0, nchunk * 128)])

        def gat(k, b):
            return pltpu.make_async_copy(g_hbm.at[giv.at[pl.ds(k * 128, 128)]],
                                         bufs[b], gsems[b])

        def wr(k, b):
            return pltpu.make_async_copy(
                bufs[b], out_hbm.at[pl.ds(base + k * 128, 128)], wsems[b])

        for b in range(3):
            gat(b, b).start()

        nfull = nchunk // 3

        def chunk3(kk, _):
            k = kk * 3
            for b in range(3):
                gat(k + b, b).wait()
                wr(k + b, b).start()

                @pl.when(kk < nfull - 1)
                def _():
                    wr(k + b, b).wait()
                    gat(k + b + 3, b).start()
            return ()

        lax.fori_loop(0, nfull, chunk3, ())
        for r in range(nchunk - nfull * 3):
            k = nfull * 3 + r
            wr(k - 3, r).wait()
            gat(k, r).start()
            gat(k, r).wait()
            wr(k, r).start()
            wr(k, r).wait()
        for b in range(nchunk - nfull * 3, 3):
            wr(nfull * 3 - 3 + b, b).wait()

    @pl.when(c == 0)
    def _():
        run(s * (_LGC0 * 128), _LGC0)

    @pl.when(c == 1)
    def _():
        run(16 * (_LGC0 * 128) + s * (_LGC1 * 128), _LGC1)


_lg_call = pl.kernel(
    _lg_body,
    out_type=jax.ShapeDtypeStruct((P, D), jnp.float32),
    mesh=_MESH,
    scratch_types=[
        pltpu.VMEM((max(_LGC0, _LGC1) * 128,), jnp.int32),
        pltpu.VMEM((128, D), jnp.float32),
        pltpu.VMEM((128, D), jnp.float32),
        pltpu.VMEM((128, D), jnp.float32),
        pltpu.SemaphoreType.DMA,
        pltpu.SemaphoreType.DMA,
        pltpu.SemaphoreType.DMA,
        pltpu.SemaphoreType.DMA,
        pltpu.SemaphoreType.DMA,
        pltpu.SemaphoreType.DMA,
    ],
)


_LPB = 2560
_LNB = P // _LPB


def _loss_body(gref, pref, oref):
    i = pl.program_id(0)
    nb = _LPB // (2 * S)
    g3 = gref[...].reshape(nb, 2 * S, D)
    d = jnp.sum(g3 * pref[...][:, None, :], axis=-1) * (1.0 / TAU)
    e = jnp.exp(d)
    ps = jnp.sum(e[:, :S], axis=-1)
    ns_ = jnp.sum(e[:, S:], axis=-1)
    term = -jnp.log(ps / (ps + LAM * ns_))
    mask = (i * nb + lax.broadcasted_iota(jnp.int32, (nb,), 0)) < N
    blocksum = jnp.sum(jnp.where(mask, term, 0.0)) * (1.0 / N)

    @pl.when(i == 0)
    def _():
        oref[...] = jnp.zeros_like(oref)

    oref[...] = oref[...] + blocksum.reshape(1, 1)


def _loss(grows, proj):
    return pl.pallas_call(
        _loss_body,
        grid=(_LNB,),
        in_specs=[
            pl.BlockSpec((_LPB, D), lambda i: (i, 0)),
            pl.BlockSpec((_LPB // (2 * S), D), lambda i: (i, 0)),
        ],
        out_specs=pl.BlockSpec((1, 1), lambda i: (0, 0)),
        out_shape=jax.ShapeDtypeStruct((1, 1), jnp.float32),
    )(grows, proj)


def kernel(x, edge_index, pos_idx, neg_idx, W0, b0, W1, b1, Wt1, bt1, gamma,
           beta, Wt2, bt2, PW0, Pb0, PW1, Pb1, PW2, Pb2):
    src = edge_index[0]
    dst = edge_index[1]
    padn = EP - E
    i32 = jnp.int32

    src_p = jnp.concatenate([src, jnp.full((padn,), N, i32)])
    dst_p = jnp.concatenate([dst, jnp.full((padn,), N, i32)])
    src2d = src_p.reshape(1280, 128)
    srcAB = jnp.concatenate([src2d, src2d + NPAD], axis=0)
    dst2d = dst_p.reshape(1280, 128)
    srcdeg = jnp.concatenate([src, jnp.full((padn,), 16383, i32)])
    dstdeg = jnp.concatenate([dst, jnp.full((padn,), 16383, i32)])

    gi = jnp.concatenate([pos_idx, neg_idx], axis=1)
    gi = jnp.concatenate([gi, jnp.zeros((NA - N, 2 * S), i32)], axis=0)
    giflat = gi.reshape(-1)

    zflat = jnp.zeros((16384,), jnp.float32)
    zrows = jnp.zeros((NACC, HF), jnp.float32)

    hs, hd = _deg_call(srcdeg, dstdeg, zflat)
    norms = _norms(hs, hd)

    y1 = _mm1(x, norms, W0)
    agg1 = _scat_call(y1.reshape(2 * NPAD, HF), srcAB, dst2d, zrows)
    y2 = _mm2(agg1, norms, b0, W1)
    agg2 = _scat_call(y2.reshape(2 * NPAD, HF), srcAB, dst2d, zrows)

    stats = _bn_stats(x, Wt1)
    MLP_emb, proj = _mlp(stats, x, Wt1, bt1, gamma, beta, Wt2, bt2,
                         PW0, Pb0, PW1, Pb1, PW2, Pb2)
    GNN_emb = _gnn(agg2, norms, b1)

    grows = _lg_call(GNN_emb, giflat)
    lossm = _loss(grows, proj)
    return (lossm[0, 0], MLP_emb)


# speedup vs baseline: 1.0664x; 1.0633x over previous
import functools

import jax
import jax.numpy as jnp
from jax import lax
from jax.experimental import pallas as pl
from jax.experimental.pallas import tpu as pltpu
from jax.experimental.pallas import tpu_sc as plsc

N = 10000
NPAD = 10016
E = 160000
EP = 163840
D = 256
HF = 128
S = 5
TAU = 0.5
LAM = 0.1
EPS = 1e-5
NACC = 10112
NA = 10240
P = NA * 2 * S
BLK = 512
GRID = 20

_MESH = plsc.VectorSubcoreMesh(core_axis_name="c", subcore_axis_name="s")


def _deg_body(srcd, dstd, zflat, out_s, out_d, idxv, histv):
    c = lax.axis_index("c")
    s = lax.axis_index("s")
    w = s * 2 + c
    ones = jnp.full((16,), 1.0, jnp.float32)

    def one_pass(edges_hbm, out_hbm):
        pltpu.sync_copy(zflat, histv)
        pltpu.sync_copy(edges_hbm.at[pl.ds(w * 5120, 5120)], idxv)

        def chunk(j, _):
            base = pl.multiple_of(j * 128, 128)
            for g in range(8):
                idx16 = idxv[pl.ds(base + g * 16, 16)]
                plsc.addupdate_scatter(histv, [idx16], ones)
            return ()

        lax.fori_loop(0, 40, chunk, ())
        pltpu.sync_copy(histv, out_hbm.at[pl.ds(w * 16384, 16384)])

    one_pass(srcd, out_s)
    one_pass(dstd, out_d)


_deg_call = pl.kernel(
    _deg_body,
    out_type=[
        jax.ShapeDtypeStruct((32 * 16384,), jnp.float32),
        jax.ShapeDtypeStruct((32 * 16384,), jnp.float32),
    ],
    mesh=_MESH,
    scratch_types=[
        pltpu.VMEM((5120,), jnp.int32),
        pltpu.VMEM((16384,), jnp.float32),
    ],
    compiler_params=pltpu.CompilerParams(needs_layout_passes=False),
)


def _norm_body(sref, dref, oref):
    sh = jnp.sum(sref[...].reshape(32, 128, 128), axis=0)
    dh = jnp.sum(dref[...].reshape(32, 128, 128), axis=0)
    oref[0] = lax.rsqrt(jnp.clip(sh, 1.0, None))
    oref[1] = lax.rsqrt(jnp.clip(dh, 1.0, None))


def _norms(hs, hd):
    return pl.pallas_call(
        _norm_body,
        out_shape=jax.ShapeDtypeStruct((2, 128, 128), jnp.float32),
    )(hs.reshape(4096, 128), hd.reshape(4096, 128))


def _rowscale(v, nref, which, i):
    r = BLK // 128
    t = nref[which, pl.ds(i * r, r), :]
    return (v.reshape(r, 128, D) * t[..., None]).reshape(BLK, D)


def _mm1_body(xref, nref, wref, wt1ref, oref, stref):
    i = pl.program_id(0)
    xb = xref[...]
    res = jnp.dot(_rowscale(xb, nref, 0, i), wref[...],
                  preferred_element_type=jnp.float32)
    oref[0] = res[:, :HF]
    oref[1] = res[:, HF:]
    u = jnp.dot(xb, wt1ref[...], preferred_element_type=jnp.float32)
    rowmask = (i * BLK + lax.broadcasted_iota(jnp.int32, (BLK, 1), 0)) < N
    u = jnp.where(rowmask, u, 0.0)

    @pl.when(i == 0)
    def _():
        stref[...] = jnp.zeros_like(stref)

    stref[0:1, :] = stref[0:1, :] + jnp.sum(u, axis=0)[None, :]
    stref[1:2, :] = stref[1:2, :] + jnp.sum(u * u, axis=0)[None, :]


def _mm1(x, norms, W0, Wt1):
    return pl.pallas_call(
        _mm1_body,
        grid=(GRID,),
        in_specs=[
            pl.BlockSpec((BLK, D), lambda i: (i, 0)),
            pl.BlockSpec((2, 128, 128), lambda i: (0, 0, 0)),
            pl.BlockSpec((D, D), lambda i: (0, 0)),
            pl.BlockSpec((D, D), lambda i: (0, 0)),
        ],
        out_specs=[
            pl.BlockSpec((2, BLK, HF), lambda i: (0, i, 0)),
            pl.BlockSpec((8, D), lambda i: (0, 0)),
        ],
        out_shape=[
            jax.ShapeDtypeStruct((2, NPAD, HF), jnp.float32),
            jax.ShapeDtypeStruct((8, D), jnp.float32),
        ],
    )(x, norms, W0, Wt1)


def _mm2_body(aref, nref, bref, wref, oref):
    i = pl.program_id(0)
    h = jnp.concatenate([aref[0], aref[1]], axis=-1)
    h = _rowscale(h, nref, 1, i) + bref[...]
    h = _rowscale(h, nref, 0, i)
    res = jnp.dot(h, wref[...], preferred_element_type=jnp.float32)
    oref[0] = res[:, :HF]
    oref[1] = res[:, HF:]


def _mm2(aggf, norms, b0, W1):
    return pl.pallas_call(
        _mm2_body,
        grid=(GRID,),
        in_specs=[
            pl.BlockSpec((2, BLK, HF), lambda i: (0, i, 0)),
            pl.BlockSpec((2, 128, 128), lambda i: (0, 0, 0)),
            pl.BlockSpec((1, D), lambda i: (0, 0)),
            pl.BlockSpec((D, D), lambda i: (0, 0)),
        ],
        out_specs=pl.BlockSpec((2, BLK, HF), lambda i: (0, i, 0)),
        out_shape=jax.ShapeDtypeStruct((2, NPAD, HF), jnp.float32),
    )(aggf.reshape(2, NACC, HF), norms, b0.reshape(1, D), W1)


def _bn_body(xref, wref, oref):
    i = pl.program_id(0)
    u = jnp.dot(xref[...], wref[...], preferred_element_type=jnp.float32)
    rowmask = (i * BLK + lax.broadcasted_iota(jnp.int32, (BLK, 1), 0)) < N
    u = jnp.where(rowmask, u, 0.0)

    @pl.when(i == 0)
    def _():
        oref[...] = jnp.zeros_like(oref)

    oref[0:1, :] = oref[0:1, :] + jnp.sum(u, axis=0)[None, :]
    oref[1:2, :] = oref[1:2, :] + jnp.sum(u * u, axis=0)[None, :]


def _bn_stats(x, Wt1):
    return pl.pallas_call(
        _bn_body,
        grid=(GRID,),
        in_specs=[
            pl.BlockSpec((BLK, D), lambda i: (i, 0)),
            pl.BlockSpec((D, D), lambda i: (0, 0)),
        ],
        out_specs=pl.BlockSpec((8, D), lambda i: (0, 0)),
        out_shape=jax.ShapeDtypeStruct((8, D), jnp.float32),
    )(x, Wt1)


def _l2n(v):
    ss = jnp.sum(v * v, axis=-1, keepdims=True)
    return v * lax.rsqrt(jnp.maximum(ss, 1e-24))


def _gnn_body(aref, nref, b1ref, gout):
    i = pl.program_id(0)
    g = jnp.concatenate([aref[0], aref[1]], axis=-1)
    g = _rowscale(g, nref, 1, i) + b1ref[...]
    gout[...] = _l2n(g)


def _gnn(aggf, norms, b1):
    return pl.pallas_call(
        _gnn_body,
        grid=(GRID,),
        in_specs=[
            pl.BlockSpec((2, BLK, HF), lambda i: (0, i, 0)),
            pl.BlockSpec((2, 128, 128), lambda i: (0, 0, 0)),
            pl.BlockSpec((1, D), lambda i: (0, 0)),
        ],
        out_specs=pl.BlockSpec((BLK, D), lambda i: (i, 0)),
        out_shape=jax.ShapeDtypeStruct((N, D), jnp.float32),
    )(aggf.reshape(2, NACC, HF), norms, b1.reshape(1, D))


def _mlp_body(stref, xref, wt1, bt1r, gr, br, wt2, bt2r,
              pw0, pb0, pw1, pb1, pw2, pb2, mout, pout):
    u = jnp.dot(xref[...], wt1[...], preferred_element_type=jnp.float32)
    s1 = stref[0:1, :]
    s2 = stref[1:2, :]
    m = s1 * (1.0 / N)
    var = s2 * (1.0 / N) - m * m
    t = (u - m) * lax.rsqrt(var + EPS) * gr[...] + br[...]
    t = jnp.maximum(t, 0.0)
    t = jnp.dot(t, wt2[...], preferred_element_type=jnp.float32) + bt2r[...]
    mlp = _l2n(t)
    mout[...] = mlp

    p = jnp.maximum(jnp.dot(mlp, pw0[...], preferred_element_type=jnp.float32) + pb0[...], 0.0)
    p = jnp.maximum(jnp.dot(p, pw1[...], preferred_element_type=jnp.float32) + pb1[...], 0.0)
    p = jnp.dot(p, pw2[...], preferred_element_type=jnp.float32) + pb2[...]
    pout[...] = _l2n(p)


def _mlp(stats, x, Wt1, bt1, gamma, beta, Wt2, bt2,
         PW0, Pb0, PW1, Pb1, PW2, Pb2):
    row = lambda i: (i, 0)
    full = lambda i: (0, 0)
    vec = pl.BlockSpec((1, D), full)
    mat = pl.BlockSpec((D, D), full)
    return pl.pallas_call(
        _mlp_body,
        grid=(GRID,),
        in_specs=[
            pl.BlockSpec((8, D), full),
            pl.BlockSpec((BLK, D), row),
            mat, vec, vec, vec, mat, vec,
            mat, vec, mat, vec, mat, vec,
        ],
        out_specs=[
            pl.BlockSpec((BLK, D), row),
            pl.BlockSpec((BLK, D), row),
        ],
        out_shape=[
            jax.ShapeDtypeStruct((N, D), jnp.float32),
            jax.ShapeDtypeStruct((N, D), jnp.float32),
        ],
    )(stats, x,
      Wt1, bt1.reshape(1, D), gamma.reshape(1, D), beta.reshape(1, D),
      Wt2, bt2.reshape(1, D), PW0, Pb0.reshape(1, D), PW1, Pb1.reshape(1, D),
      PW2, Pb2.reshape(1, D))


def _scat_body(y_hbm, sab_hbm, dst_hbm, z_hbm, out_hbm, sidx, didx,
               gbuf0, gbuf1, acc, gsem0, gsem1):
    c = lax.axis_index("c")
    s = lax.axis_index("s")
    pltpu.sync_copy(z_hbm.at[pl.ds(s * 632, 632)], acc.at[pl.ds(s * 632, 632)])
    plsc.subcore_barrier()

    def gat(j, buf, sem):
        return pltpu.make_async_copy(y_hbm.at[sidx.at[j]], buf, sem)

    def phase(ph, _):
        pltpu.sync_copy(sab_hbm.at[pl.ds(c * 1280 + s * 80 + ph * 40, 40)], sidx)
        pltpu.sync_copy(dst_hbm.at[pl.ds(s * 80 + ph * 40, 40)], didx)
        gat(0, gbuf0, gsem0).start()

        def chunk2(jj, _):
            j = jj * 2
            gat(j + 1, gbuf1, gsem1).start()
            gat(j, gbuf0, gsem0).wait()
            pltpu.sync_copy(gbuf0, acc.at[didx.at[j]], add=True)

            @pl.when(jj < 19)
            def _():
                gat(j + 2, gbuf0, gsem0).start()

            gat(j + 1, gbuf1, gsem1).wait()
            pltpu.sync_copy(gbuf1, acc.at[didx.at[j + 1]], add=True)
            return ()

        lax.fori_loop(0, 20, chunk2, ())
        return ()

    lax.fori_loop(0, 2, phase, ())
    plsc.subcore_barrier()
    pltpu.sync_copy(acc.at[pl.ds(s * 632, 632)],
                    out_hbm.at[pl.ds(c * NACC + s * 632, 632)])


_scat_call = pl.kernel(
    _scat_body,
    out_type=jax.ShapeDtypeStruct((2 * NACC, HF), jnp.float32),
    mesh=_MESH,
    scratch_types=[
        pltpu.VMEM((40, 128), jnp.int32),
        pltpu.VMEM((40, 128), jnp.int32),
        pltpu.VMEM((128, HF), jnp.float32),
        pltpu.VMEM((128, HF), jnp.float32),
        pltpu.VMEM_SHARED((NACC, HF), jnp.float32),
        pltpu.SemaphoreType.DMA,
        pltpu.SemaphoreType.DMA,
    ],
)


_LGC0 = 25
_LGC1 = 50 - _LGC0


def _lg_body(g_hbm, gi_hbm, out_hbm, giv, gbuf0, gbuf1, gbuf2,
             gs0, gs1, gs2, ws0, ws1, ws2):
    c = lax.axis_index("c")
    s = lax.axis_index("s")

    bufs = (gbuf0, gbuf1, gbuf2)
    gsems = (gs0, gs1, gs2)
    wsems = (ws0, ws1, ws2)

    def run(base, nchunk):
        pltpu.sync_copy(gi_hbm.at[pl.ds(base, nchunk * 128)],
                        giv.at[pl.ds(0, nchunk * 128)])

        def gat(k, b):
            return pltpu.make_async_copy(g_hbm.at[giv.at[pl.ds(k * 128, 128)]],
                                         bufs[b], gsems[b])

        def wr(k, b):
            return pltpu.make_async_copy(
                bufs[b], out_hbm.at[pl.ds(base + k * 128, 128)], wsems[b])

        for b in range(3):
            gat(b, b).start()

        nfull = nchunk // 3

        def chunk3(kk, _):
            k = kk * 3
            for b in range(3):
                gat(k + b, b).wait()
                wr(k + b, b).start()

                @pl.when(kk < nfull - 1)
                def _():
                    wr(k + b, b).wait()
                    gat(k + b + 3, b).start()
            return ()

        lax.fori_loop(0, nfull, chunk3, ())
        for r in range(nchunk - nfull * 3):
            k = nfull * 3 + r
            wr(k - 3, r).wait()
            gat(k, r).start()
            gat(k, r).wait()
            wr(k, r).start()
            wr(k, r).wait()
        for b in range(nchunk - nfull * 3, 3):
            wr(nfull * 3 - 3 + b, b).wait()

    @pl.when(c == 0)
    def _():
        run(s * (_LGC0 * 128), _LGC0)

    @pl.when(c == 1)
    def _():
        run(16 * (_LGC0 * 128) + s * (_LGC1 * 128), _LGC1)


_lg_call = pl.kernel(
    _lg_body,
    out_type=jax.ShapeDtypeStruct((P, D), jnp.float32),
    mesh=_MESH,
    scratch_types=[
        pltpu.VMEM((max(_LGC0, _LGC1) * 128,), jnp.int32),
        pltpu.VMEM((128, D), jnp.float32),
        pltpu.VMEM((128, D), jnp.float32),
        pltpu.VMEM((128, D), jnp.float32),
        pltpu.SemaphoreType.DMA,
        pltpu.SemaphoreType.DMA,
        pltpu.SemaphoreType.DMA,
        pltpu.SemaphoreType.DMA,
        pltpu.SemaphoreType.DMA,
        pltpu.SemaphoreType.DMA,
    ],
)


_LPB = 2560
_LNB = P // _LPB


def _loss_body(gref, pref, oref):
    i = pl.program_id(0)
    nb = _LPB // (2 * S)
    g3 = gref[...].reshape(nb, 2 * S, D)
    d = jnp.sum(g3 * pref[...][:, None, :], axis=-1) * (1.0 / TAU)
    e = jnp.exp(d)
    ps = jnp.sum(e[:, :S], axis=-1)
    ns_ = jnp.sum(e[:, S:], axis=-1)
    term = -jnp.log(ps / (ps + LAM * ns_))
    mask = (i * nb + lax.broadcasted_iota(jnp.int32, (nb,), 0)) < N
    blocksum = jnp.sum(jnp.where(mask, term, 0.0)) * (1.0 / N)

    @pl.when(i == 0)
    def _():
        oref[...] = jnp.zeros_like(oref)

    oref[...] = oref[...] + blocksum.reshape(1, 1)


def _loss(grows, proj):
    return pl.pallas_call(
        _loss_body,
        grid=(_LNB,),
        in_specs=[
            pl.BlockSpec((_LPB, D), lambda i: (i, 0)),
            pl.BlockSpec((_LPB // (2 * S), D), lambda i: (i, 0)),
        ],
        out_specs=pl.BlockSpec((1, 1), lambda i: (0, 0)),
        out_shape=jax.ShapeDtypeStruct((1, 1), jnp.float32),
    )(grows, proj)


def kernel(x, edge_index, pos_idx, neg_idx, W0, b0, W1, b1, Wt1, bt1, gamma,
           beta, Wt2, bt2, PW0, Pb0, PW1, Pb1, PW2, Pb2):
    src = edge_index[0]
    dst = edge_index[1]
    padn = EP - E
    i32 = jnp.int32

    src_p = jnp.concatenate([src, jnp.full((padn,), N, i32)])
    dst_p = jnp.concatenate([dst, jnp.full((padn,), N, i32)])
    src2d = src_p.reshape(1280, 128)
    srcAB = jnp.concatenate([src2d, src2d + NPAD], axis=0)
    dst2d = dst_p.reshape(1280, 128)
    srcdeg = jnp.concatenate([src, jnp.full((padn,), 16383, i32)])
    dstdeg = jnp.concatenate([dst, jnp.full((padn,), 16383, i32)])

    gi = jnp.concatenate([pos_idx, neg_idx], axis=1)
    gi = jnp.concatenate([gi, jnp.zeros((NA - N, 2 * S), i32)], axis=0)
    giflat = gi.reshape(-1)

    zflat = jnp.zeros((16384,), jnp.float32)
    zrows = jnp.zeros((NACC, HF), jnp.float32)

    hs, hd = _deg_call(srcdeg, dstdeg, zflat)
    norms = _norms(hs, hd)

    y1, stats = _mm1(x, norms, W0, Wt1)
    agg1 = _scat_call(y1.reshape(2 * NPAD, HF), srcAB, dst2d, zrows)
    y2 = _mm2(agg1, norms, b0, W1)
    agg2 = _scat_call(y2.reshape(2 * NPAD, HF), srcAB, dst2d, zrows)

    MLP_emb, proj = _mlp(stats, x, Wt1, bt1, gamma, beta, Wt2, bt2,
                         PW0, Pb0, PW1, Pb1, PW2, Pb2)
    GNN_emb = _gnn(agg2, norms, b1)

    grows = _lg_call(GNN_emb, giflat)
    lossm = _loss(grows, proj)
    return (lossm[0, 0], MLP_emb)


# speedup vs baseline: 1.0676x; 1.0011x over previous
import jax
import jax.numpy as jnp
from jax import lax
from jax.experimental import pallas as pl
from jax.experimental.pallas import tpu as pltpu
from jax.experimental.pallas import tpu_sc as plsc

N = 10000
NPAD = 10016
E = 160000
EP = 163840
D = 256
HF = 128
S = 5
TAU = 0.5
LAM = 0.1
EPS = 1e-5
NACC = 10112
NA = 10240
P = NA * 2 * S
BLK = 512
GRID = 20

_MESH = plsc.VectorSubcoreMesh(core_axis_name="c", subcore_axis_name="s")


def _deg_body(srcd, dstd, zflat, out_s, out_d, idxv, histv):
    c = lax.axis_index("c")
    s = lax.axis_index("s")
    w = s * 2 + c
    ones = jnp.full((16,), 1.0, jnp.float32)

    def one_pass(edges_hbm, out_hbm):
        pltpu.sync_copy(zflat, histv)
        pltpu.sync_copy(edges_hbm.at[pl.ds(w * 5120, 5120)], idxv)

        def chunk(j, _):
            base = pl.multiple_of(j * 128, 128)
            for g in range(8):
                idx16 = idxv[pl.ds(base + g * 16, 16)]
                plsc.addupdate_scatter(histv, [idx16], ones)
            return ()

        lax.fori_loop(0, 40, chunk, ())
        pltpu.sync_copy(histv, out_hbm.at[pl.ds(w * 16384, 16384)])

    one_pass(srcd, out_s)
    one_pass(dstd, out_d)


_deg_call = pl.kernel(
    _deg_body,
    out_type=[
        jax.ShapeDtypeStruct((32 * 16384,), jnp.float32),
        jax.ShapeDtypeStruct((32 * 16384,), jnp.float32),
    ],
    mesh=_MESH,
    scratch_types=[
        pltpu.VMEM((5120,), jnp.int32),
        pltpu.VMEM((16384,), jnp.float32),
    ],
    compiler_params=pltpu.CompilerParams(needs_layout_passes=False),
)


def _norm_body(sref, dref, oref):
    sh = jnp.sum(sref[...].reshape(32, 128, 128), axis=0)
    dh = jnp.sum(dref[...].reshape(32, 128, 128), axis=0)
    oref[0] = lax.rsqrt(jnp.clip(sh, 1.0, None))
    oref[1] = lax.rsqrt(jnp.clip(dh, 1.0, None))


def _norms(hs, hd):
    return pl.pallas_call(
        _norm_body,
        out_shape=jax.ShapeDtypeStruct((2, 128, 128), jnp.float32),
    )(hs.reshape(4096, 128), hd.reshape(4096, 128))


def _rowscale(v, nref, which, i):
    r = BLK // 128
    t = nref[which, pl.ds(i * r, r), :]
    return (v.reshape(r, 128, D) * t[..., None]).reshape(BLK, D)


def _mm1_body(xref, nref, wref, wt1ref, oref, stref):
    i = pl.program_id(0)
    xb = xref[...]
    res = jnp.dot(_rowscale(xb, nref, 0, i), wref[...],
                  preferred_element_type=jnp.float32)
    oref[0] = res[:, :HF]
    oref[1] = res[:, HF:]
    u = jnp.dot(xb, wt1ref[...], preferred_element_type=jnp.float32)
    rowmask = (i * BLK + lax.broadcasted_iota(jnp.int32, (BLK, 1), 0)) < N
    u = jnp.where(rowmask, u, 0.0)

    @pl.when(i == 0)
    def _():
        stref[...] = jnp.zeros_like(stref)

    stref[0:1, :] = stref[0:1, :] + jnp.sum(u, axis=0)[None, :]
    stref[1:2, :] = stref[1:2, :] + jnp.sum(u * u, axis=0)[None, :]


def _mm1(x, norms, W0, Wt1):
    return pl.pallas_call(
        _mm1_body,
        grid=(GRID,),
        in_specs=[
            pl.BlockSpec((BLK, D), lambda i: (i, 0)),
            pl.BlockSpec((2, 128, 128), lambda i: (0, 0, 0)),
            pl.BlockSpec((D, D), lambda i: (0, 0)),
            pl.BlockSpec((D, D), lambda i: (0, 0)),
        ],
        out_specs=[
            pl.BlockSpec((2, BLK, HF), lambda i: (0, i, 0)),
            pl.BlockSpec((8, D), lambda i: (0, 0)),
        ],
        out_shape=[
            jax.ShapeDtypeStruct((2, NPAD, HF), jnp.float32),
            jax.ShapeDtypeStruct((8, D), jnp.float32),
        ],
    )(x, norms, W0, Wt1)


def _mm2_body(aref, nref, bref, wref, oref):
    i = pl.program_id(0)
    h = jnp.concatenate([aref[0], aref[1]], axis=-1)
    h = _rowscale(h, nref, 1, i) + bref[...]
    h = _rowscale(h, nref, 0, i)
    res = jnp.dot(h, wref[...], preferred_element_type=jnp.float32)
    oref[0] = res[:, :HF]
    oref[1] = res[:, HF:]


def _mm2(aggf, norms, b0, W1):
    return pl.pallas_call(
        _mm2_body,
        grid=(GRID,),
        in_specs=[
            pl.BlockSpec((2, BLK, HF), lambda i: (0, i, 0)),
            pl.BlockSpec((2, 128, 128), lambda i: (0, 0, 0)),
            pl.BlockSpec((1, D), lambda i: (0, 0)),
            pl.BlockSpec((D, D), lambda i: (0, 0)),
        ],
        out_specs=pl.BlockSpec((2, BLK, HF), lambda i: (0, i, 0)),
        out_shape=jax.ShapeDtypeStruct((2, NPAD, HF), jnp.float32),
    )(aggf.reshape(2, NACC, HF), norms, b0.reshape(1, D), W1)


def _l2n(v):
    ss = jnp.sum(v * v, axis=-1, keepdims=True)
    return v * lax.rsqrt(jnp.maximum(ss, 1e-24))


def _gnn_body(aref, nref, b1ref, gout):
    i = pl.program_id(0)
    g = jnp.concatenate([aref[0], aref[1]], axis=-1)
    g = _rowscale(g, nref, 1, i) + b1ref[...]
    gout[...] = _l2n(g)


def _gnn(aggf, norms, b1):
    return pl.pallas_call(
        _gnn_body,
        grid=(GRID,),
        in_specs=[
            pl.BlockSpec((2, BLK, HF), lambda i: (0, i, 0)),
            pl.BlockSpec((2, 128, 128), lambda i: (0, 0, 0)),
            pl.BlockSpec((1, D), lambda i: (0, 0)),
        ],
        out_specs=pl.BlockSpec((BLK, D), lambda i: (i, 0)),
        out_shape=jax.ShapeDtypeStruct((N, D), jnp.float32),
    )(aggf.reshape(2, NACC, HF), norms, b1.reshape(1, D))


def _mlp_body(stref, xref, wt1, bt1r, gr, br, wt2, bt2r,
              pw0, pb0, pw1, pb1, pw2, pb2, mout, pout):
    u = jnp.dot(xref[...], wt1[...], preferred_element_type=jnp.float32)
    s1 = stref[0:1, :]
    s2 = stref[1:2, :]
    m = s1 * (1.0 / N)
    var = s2 * (1.0 / N) - m * m
    t = (u - m) * lax.rsqrt(var + EPS) * gr[...] + br[...]
    t = jnp.maximum(t, 0.0)
    t = jnp.dot(t, wt2[...], preferred_element_type=jnp.float32) + bt2r[...]
    mlp = _l2n(t)
    mout[...] = mlp

    p = jnp.maximum(jnp.dot(mlp, pw0[...], preferred_element_type=jnp.float32) + pb0[...], 0.0)
    p = jnp.maximum(jnp.dot(p, pw1[...], preferred_element_type=jnp.float32) + pb1[...], 0.0)
    p = jnp.dot(p, pw2[...], preferred_element_type=jnp.float32) + pb2[...]
    pout[...] = _l2n(p)


def _mlp(stats, x, Wt1, bt1, gamma, beta, Wt2, bt2,
         PW0, Pb0, PW1, Pb1, PW2, Pb2):
    row = lambda i: (i, 0)
    full = lambda i: (0, 0)
    vec = pl.BlockSpec((1, D), full)
    mat = pl.BlockSpec((D, D), full)
    return pl.pallas_call(
        _mlp_body,
        grid=(GRID,),
        in_specs=[
            pl.BlockSpec((8, D), full),
            pl.BlockSpec((BLK, D), row),
            mat, vec, vec, vec, mat, vec,
            mat, vec, mat, vec, mat, vec,
        ],
        out_specs=[
            pl.BlockSpec((BLK, D), row),
            pl.BlockSpec((BLK, D), row),
        ],
        out_shape=[
            jax.ShapeDtypeStruct((N, D), jnp.float32),
            jax.ShapeDtypeStruct((N, D), jnp.float32),
        ],
    )(stats, x,
      Wt1, bt1.reshape(1, D), gamma.reshape(1, D), beta.reshape(1, D),
      Wt2, bt2.reshape(1, D), PW0, Pb0.reshape(1, D), PW1, Pb1.reshape(1, D),
      PW2, Pb2.reshape(1, D))


def _scat_body(y_hbm, sab_hbm, dst_hbm, z_hbm, out_hbm, sidx, didx,
               gbuf0, gbuf1, acc, gsem0, gsem1):
    c = lax.axis_index("c")
    s = lax.axis_index("s")
    pltpu.sync_copy(z_hbm.at[pl.ds(s * 632, 632)], acc.at[pl.ds(s * 632, 632)])
    plsc.subcore_barrier()

    def gat(j, buf, sem):
        return pltpu.make_async_copy(y_hbm.at[sidx.at[j]], buf, sem)

    def phase(ph, _):
        pltpu.sync_copy(sab_hbm.at[pl.ds(c * 1280 + s * 80 + ph * 40, 40)], sidx)
        pltpu.sync_copy(dst_hbm.at[pl.ds(s * 80 + ph * 40, 40)], didx)
        gat(0, gbuf0, gsem0).start()

        def chunk2(jj, _):
            j = jj * 2
            gat(j + 1, gbuf1, gsem1).start()
            gat(j, gbuf0, gsem0).wait()
            pltpu.sync_copy(gbuf0, acc.at[didx.at[j]], add=True)

            @pl.when(jj < 19)
            def _():
                gat(j + 2, gbuf0, gsem0).start()

            gat(j + 1, gbuf1, gsem1).wait()
            pltpu.sync_copy(gbuf1, acc.at[didx.at[j + 1]], add=True)
            return ()

        lax.fori_loop(0, 20, chunk2, ())
        return ()

    lax.fori_loop(0, 2, phase, ())
    plsc.subcore_barrier()
    pltpu.sync_copy(acc.at[pl.ds(s * 632, 632)],
                    out_hbm.at[pl.ds(c * NACC + s * 632, 632)])


_scat_call = pl.kernel(
    _scat_body,
    out_type=jax.ShapeDtypeStruct((2 * NACC, HF), jnp.float32),
    mesh=_MESH,
    scratch_types=[
        pltpu.VMEM((40, 128), jnp.int32),
        pltpu.VMEM((40, 128), jnp.int32),
        pltpu.VMEM((128, HF), jnp.float32),
        pltpu.VMEM((128, HF), jnp.float32),
        pltpu.VMEM_SHARED((NACC, HF), jnp.float32),
        pltpu.SemaphoreType.DMA,
        pltpu.SemaphoreType.DMA,
    ],
)


_LGC0 = 25
_LGC1 = 50 - _LGC0


def _lg_body(g_hbm, gi_hbm, out_hbm, giv, gbuf0, gbuf1, gbuf2,
             gs0, gs1, gs2, ws0, ws1, ws2):
    c = lax.axis_index("c")
    s = lax.axis_index("s")

    bufs = (gbuf0, gbuf1, gbuf2)
    gsems = (gs0, gs1, gs2)
    wsems = (ws0, ws1, ws2)

    def run(base, nchunk):
        pltpu.sync_copy(gi_hbm.at[pl.ds(base, nchunk * 128)],
                        giv.at[pl.ds(0, nchunk * 128)])

        def gat(k, b):
            return pltpu.make_async_copy(g_hbm.at[giv.at[pl.ds(k * 128, 128)]],
                                         bufs[b], gsems[b])

        def wr(k, b):
            return pltpu.make_async_copy(
                bufs[b], out_hbm.at[pl.ds(base + k * 128, 128)], wsems[b])

        for b in range(3):
            gat(b, b).start()

        nfull = nchunk // 3

        def chunk3(kk, _):
            k = kk * 3
            for b in range(3):
                gat(k + b, b).wait()
                wr(k + b, b).start()

                @pl.when(kk < nfull - 1)
                def _():
                    wr(k + b, b).wait()
                    gat(k + b + 3, b).start()
            return ()

        lax.fori_loop(0, nfull, chunk3, ())
        for r in range(nchunk - nfull * 3):
            k = nfull * 3 + r
            wr(k - 3, r).wait()
            gat(k, r).start()
            gat(k, r).wait()
            wr(k, r).start()
            wr(k, r).wait()
        for b in range(nchunk - nfull * 3, 3):
            wr(nfull * 3 - 3 + b, b).wait()

    @pl.when(c == 0)
    def _():
        run(s * (_LGC0 * 128), _LGC0)

    @pl.when(c == 1)
    def _():
        run(16 * (_LGC0 * 128) + s * (_LGC1 * 128), _LGC1)


_lg_call = pl.kernel(
    _lg_body,
    out_type=jax.ShapeDtypeStruct((P, D), jnp.float32),
    mesh=_MESH,
    scratch_types=[
        pltpu.VMEM((max(_LGC0, _LGC1) * 128,), jnp.int32),
        pltpu.VMEM((128, D), jnp.float32),
        pltpu.VMEM((128, D), jnp.float32),
        pltpu.VMEM((128, D), jnp.float32),
        pltpu.SemaphoreType.DMA,
        pltpu.SemaphoreType.DMA,
        pltpu.SemaphoreType.DMA,
        pltpu.SemaphoreType.DMA,
        pltpu.SemaphoreType.DMA,
        pltpu.SemaphoreType.DMA,
    ],
)


_LPB = 2560
_LNB = P // _LPB


def _loss_body(gref, pref, oref):
    i = pl.program_id(0)
    nb = _LPB // (2 * S)
    g3 = gref[...].reshape(nb, 2 * S, D)
    d = jnp.sum(g3 * pref[...][:, None, :], axis=-1) * (1.0 / TAU)
    e = jnp.exp(d)
    ps = jnp.sum(e[:, :S], axis=-1)
    ns_ = jnp.sum(e[:, S:], axis=-1)
    term = -jnp.log(ps / (ps + LAM * ns_))
    mask = (i * nb + lax.broadcasted_iota(jnp.int32, (nb,), 0)) < N
    blocksum = jnp.sum(jnp.where(mask, term, 0.0)) * (1.0 / N)

    @pl.when(i == 0)
    def _():
        oref[...] = jnp.zeros_like(oref)

    oref[...] = oref[...] + blocksum.reshape(1, 1)


def _loss(grows, proj):
    return pl.pallas_call(
        _loss_body,
        grid=(_LNB,),
        in_specs=[
            pl.BlockSpec((_LPB, D), lambda i: (i, 0)),
            pl.BlockSpec((_LPB // (2 * S), D), lambda i: (i, 0)),
        ],
        out_specs=pl.BlockSpec((1, 1), lambda i: (0, 0)),
        out_shape=jax.ShapeDtypeStruct((1, 1), jnp.float32),
    )(grows, proj)


def kernel(x, edge_index, pos_idx, neg_idx, W0, b0, W1, b1, Wt1, bt1, gamma,
           beta, Wt2, bt2, PW0, Pb0, PW1, Pb1, PW2, Pb2):
    src = edge_index[0]
    dst = edge_index[1]
    padn = EP - E
    i32 = jnp.int32

    src_p = jnp.concatenate([src, jnp.full((padn,), N, i32)])
    dst_p = jnp.concatenate([dst, jnp.full((padn,), N, i32)])
    src2d = src_p.reshape(1280, 128)
    srcAB = jnp.concatenate([src2d, src2d + NPAD], axis=0)
    dst2d = dst_p.reshape(1280, 128)
    srcdeg = jnp.concatenate([src, jnp.full((padn,), 16383, i32)])
    dstdeg = jnp.concatenate([dst, jnp.full((padn,), 16383, i32)])

    gi = jnp.concatenate([pos_idx, neg_idx], axis=1)
    gi = jnp.concatenate([gi, jnp.zeros((NA - N, 2 * S), i32)], axis=0)
    giflat = gi.reshape(-1)

    zflat = jnp.zeros((16384,), jnp.float32)
    zrows = jnp.zeros((NACC, HF), jnp.float32)

    hs, hd = _deg_call(srcdeg, dstdeg, zflat)
    norms = _norms(hs, hd)

    y1, stats = _mm1(x, norms, W0, Wt1)
    agg1 = _scat_call(y1.reshape(2 * NPAD, HF), srcAB, dst2d, zrows)
    y2 = _mm2(agg1, norms, b0, W1)
    agg2 = _scat_call(y2.reshape(2 * NPAD, HF), srcAB, dst2d, zrows)

    MLP_emb, proj = _mlp(stats, x, Wt1, bt1, gamma, beta, Wt2, bt2,
                         PW0, Pb0, PW1, Pb1, PW2, Pb2)
    GNN_emb = _gnn(agg2, norms, b1)

    grows = _lg_call(GNN_emb, giflat)
    lossm = _loss(grows, proj)
    return (lossm[0, 0], MLP_emb)
